# baseline (speedup 1.0000x reference)
import jax
import jax.numpy as jnp
from jax import lax
from jax.experimental import pallas as pl

_DEPTH = 1000
_L = 20
_BS = 8


def _body(x_ref, o_ref):
    x = x_ref[...]
    d = lax.broadcasted_iota(jnp.int32, (_BS, _DEPTH, _L), 1)
    o_ref[...] = (d == x[:, None, :]).astype(jnp.float32)


def kernel(X_in, ones):
    del ones
    B, L = X_in.shape
    return pl.pallas_call(
        _body,
        grid=(B // _BS,),
        in_specs=[pl.BlockSpec((_BS, L), lambda i: (i, 0))],
        out_specs=pl.BlockSpec((_BS, _DEPTH, L), lambda i: (i, 0, 0)),
        out_shape=jax.ShapeDtypeStruct((B, _DEPTH, L), jnp.float32),
    )(X_in)
